# Initial kernel scaffold; baseline (speedup 1.0000x reference)
import jax
import jax.numpy as jnp
from jax.experimental import pallas as pl


def kernel(seq, seq_pos, seq_iter, seq_mass_forward, candidate_aa, candidate_aa_mass, seq_table, pos_table, iter_table):
    raise NotImplementedError("write your pallas kernel here")


# speedup vs baseline: 1.6663x; 1.6663x over previous
import jax
import jax.numpy as jnp
import numpy as np
from jax.experimental import pallas as pl

B = 1024
L = 20
C = 23
HID = 128
NH = 4
DH = HID // NH
VOCAB = 23
POS = 200
ITER = 1000
LMAX = 10000.0
LMIN = 0.001

BB = 8
LP = 24
LC = L * C

_INTERPRET = False


def _div_term_np():
    base = LMAX / (2.0 * np.pi)
    scale = LMIN / LMAX
    return (base * scale ** (np.arange(0, DH, 2, dtype=np.float32) / DH)).astype(np.float32)


def _inv_dt128_np():
    dt = _div_term_np()
    inv = (np.float32(1.0) / dt).astype(np.float32)
    h = (inv / 2.0).astype(np.float32)
    return np.concatenate([inv, inv, inv, inv, h, h, h, h]).reshape(1, HID)


def _onehot(idx, n):
    iota = jax.lax.broadcasted_iota(jnp.int32, (1, n), 1)
    return (idx == iota).astype(jnp.float32)


def _body(sqF_r, spF_r, siF_r, aF_r, caF_r, arep_r, cbt_r, st_r, pt_r, it_r, dtB_r,
          seq_e_r, seq_me_r, cand_e_r, cand_me_r):
    st = st_r[...]
    dtB3 = dtB_r[...][None]
    lane = jax.lax.broadcasted_iota(jnp.int32, (1, 1, HID), 2)
    sin_mask = (lane % DH) < (DH // 2)

    sp = jnp.minimum(spF_r[...], POS - 1)
    si = jnp.minimum(siF_r[...], ITER - 1)
    pi_flat = (jnp.dot(_onehot(sp, POS), pt_r[...], preferred_element_type=jnp.float32)
               + jnp.dot(_onehot(si, ITER), it_r[...], preferred_element_type=jnp.float32))
    es_flat = jnp.dot(_onehot(sqF_r[...], VOCAB), st, preferred_element_type=jnp.float32)
    sc_flat = jnp.dot(_onehot(caF_r[...], VOCAB), st, preferred_element_type=jnp.float32)

    j_iota = jax.lax.broadcasted_iota(jnp.int32, (LC, LP), 0)
    k_iota = jax.lax.broadcasted_iota(jnp.int32, (LC, LP), 1)
    Rl = ((j_iota // C) == k_iota).astype(jnp.float32)
    Rc = ((j_iota % C) == k_iota).astype(jnp.float32)

    aF = aF_r[...]

    for b in range(BB):
        pb = pi_flat[b * LP:(b + 1) * LP]
        eb = es_flat[b * LP:(b + 1) * LP]
        scb = sc_flat[b * LP:(b + 1) * LP]
        seq_e_r[b] = (eb + pb)[0:L]
        cand_e_r[b] = (jnp.dot(Rl, pb, preferred_element_type=jnp.float32)
                       + jnp.dot(Rc, scb, preferred_element_type=jnp.float32))

        xs = aF[b * LP:(b + 1) * LP] * dtB3[0]
        outs = jnp.where(sin_mask[0], jnp.sin(xs), jnp.cos(xs))[0:L]
        for h in range(NH):
            seq_me_r[b, :, h, :] = outs[:, h * DH:(h + 1) * DH]

    cm = arep_r[...] + cbt_r[...]
    xc = cm[:, :, None] * dtB3
    outc = jnp.where(sin_mask, jnp.sin(xc), jnp.cos(xc))
    for h in range(NH):
        cand_me_r[:, :, h, :] = outc[:, :, h * DH:(h + 1) * DH]


@jax.jit
def _run(sqF, spF, siF, aF, caF, arep, cbt, seq_table, pos_table, iter_table, dtB):
    grid = (B // BB,)
    out_shapes = [
        jax.ShapeDtypeStruct((B, L, HID), jnp.float32),
        jax.ShapeDtypeStruct((B, L, NH, DH), jnp.float32),
        jax.ShapeDtypeStruct((B, LC, HID), jnp.float32),
        jax.ShapeDtypeStruct((B, LC, NH, DH), jnp.float32),
    ]
    flat = pl.BlockSpec((BB * LP, 1), lambda i: (i, 0))
    in_specs = [
        flat, flat, flat, flat, flat,
        pl.BlockSpec((BB, LC), lambda i: (i, 0)),
        pl.BlockSpec((BB, LC), lambda i: (i, 0)),
        pl.BlockSpec((VOCAB, HID), lambda i: (0, 0)),
        pl.BlockSpec((POS, HID), lambda i: (0, 0)),
        pl.BlockSpec((ITER, HID), lambda i: (0, 0)),
        pl.BlockSpec((1, HID), lambda i: (0, 0)),
    ]
    out_specs = [
        pl.BlockSpec((BB, L, HID), lambda i: (i, 0, 0)),
        pl.BlockSpec((BB, L, NH, DH), lambda i: (i, 0, 0, 0)),
        pl.BlockSpec((BB, LC, HID), lambda i: (i, 0, 0)),
        pl.BlockSpec((BB, LC, NH, DH), lambda i: (i, 0, 0, 0)),
    ]
    return pl.pallas_call(
        _body,
        grid=grid,
        in_specs=in_specs,
        out_specs=out_specs,
        out_shape=out_shapes,
        interpret=_INTERPRET,
    )(sqF, spF, siF, aF, caF, arep, cbt, seq_table, pos_table, iter_table, dtB)


def _pad_flat(x2d, dtype):
    xp = jnp.pad(x2d.astype(dtype), ((0, 0), (0, LP - x2d.shape[1])))
    return xp.reshape(B * LP, 1)


def kernel(seq, seq_pos, seq_iter, seq_mass_forward, candidate_aa, candidate_aa_mass,
           seq_table, pos_table, iter_table):
    dtB = jnp.asarray(_inv_dt128_np())
    sqF = _pad_flat(seq, jnp.int32)
    spF = _pad_flat(seq_pos, jnp.int32)
    siF = _pad_flat(seq_iter, jnp.int32)
    aF = _pad_flat(seq_mass_forward, jnp.float32)
    caF = _pad_flat(candidate_aa, jnp.int32)
    arep = jnp.repeat(seq_mass_forward, C, axis=1)
    cbt = jnp.tile(candidate_aa_mass, (1, L))
    return tuple(_run(sqF, spF, siF, aF, caF, arep, cbt,
                      seq_table, pos_table, iter_table, dtB))
